# baseline (speedup 1.0000x reference)
import functools

import jax
import jax.numpy as jnp
from jax import lax
from jax.experimental import pallas as pl
from jax.experimental.pallas import tpu as pltpu
from jax.experimental.pallas import tpu_sc as plsc

NN = 10000
NP = 10240
D = 128
NCLS = 16
NGRAPH = 64
E = 320000
NTILE = 16
CH = 32
GRP = 8
NCHUNK = 640
NGROUP = NCHUNK // GRP
EPT = NCHUNK * CH
DNW = NP // 128
NSC = 10112
ROWS_PT = NSC // NTILE


def _leaky(x):
    return jnp.where(x > 0, x, 0.2 * x)


def _elu(x):
    return jnp.where(x > 0, x, jnp.exp(jnp.minimum(x, 0.0)) - 1.0)


RB = 2048
NRB = NP // RB


def _dense1_body(x_ref, w_ref, av_ref, ad_ref, h_ref, asv_ref, adv_ref):
    h = jnp.dot(x_ref[0], w_ref[...], preferred_element_type=jnp.float32)
    h_ref[0] = h
    asv_ref[0] = jnp.sum(h * av_ref[...], axis=-1, keepdims=True)
    adv_ref[0] = jnp.sum(h * ad_ref[...], axis=-1, keepdims=True)


def _tc_dense1(xs, W, av, ad):
    return pl.pallas_call(
        _dense1_body,
        grid=(2, NRB),
        in_specs=[
            pl.BlockSpec((1, RB, D), lambda b, r: (b, r, 0)),
            pl.BlockSpec((D, D), lambda b, r: (0, 0)),
            pl.BlockSpec((1, D), lambda b, r: (0, 0)),
            pl.BlockSpec((1, D), lambda b, r: (0, 0)),
        ],
        out_specs=[
            pl.BlockSpec((1, RB, D), lambda b, r: (b, r, 0)),
            pl.BlockSpec((1, RB, 1), lambda b, r: (b, r, 0)),
            pl.BlockSpec((1, RB, 1), lambda b, r: (b, r, 0)),
        ],
        out_shape=[
            jax.ShapeDtypeStruct((2, NP, D), jnp.float32),
            jax.ShapeDtypeStruct((2, NP, 1), jnp.float32),
            jax.ShapeDtypeStruct((2, NP, 1), jnp.float32),
        ],
    )(xs, W, av, ad)


def _mid_body(num_ref, den_ref, asv_ref, adv_ref, h_ref, b1_ref, w_ref,
              av_ref, ad_ref, h2_ref, asv2_ref, adv2_ref):
    el = jnp.exp(_leaky(asv_ref[0] + adv_ref[0]))
    den = jnp.sum(den_ref[0], axis=0)
    x = (num_ref[0] + el * h_ref[0]) / (den + el + 1e-16) + b1_ref[...]
    x = _elu(x)
    h2 = jnp.dot(x, w_ref[...], preferred_element_type=jnp.float32)
    h2_ref[0] = h2
    asv2_ref[0] = jnp.sum(h2 * av_ref[...], axis=-1, keepdims=True)
    adv2_ref[0] = jnp.sum(h2 * ad_ref[...], axis=-1, keepdims=True)


def _tc_mid(numer, denom, asv, adv, h, b1, W, av, ad):
    return pl.pallas_call(
        _mid_body,
        grid=(2, NRB),
        in_specs=[
            pl.BlockSpec((1, RB, D), lambda b, r: (b, r, 0)),
            pl.BlockSpec((1, NTILE, RB, 1), lambda b, r: (b, 0, r, 0)),
            pl.BlockSpec((1, RB, 1), lambda b, r: (b, r, 0)),
            pl.BlockSpec((1, RB, 1), lambda b, r: (b, r, 0)),
            pl.BlockSpec((1, RB, D), lambda b, r: (b, r, 0)),
            pl.BlockSpec((1, D), lambda b, r: (0, 0)),
            pl.BlockSpec((D, D), lambda b, r: (0, 0)),
            pl.BlockSpec((1, D), lambda b, r: (0, 0)),
            pl.BlockSpec((1, D), lambda b, r: (0, 0)),
        ],
        out_specs=[
            pl.BlockSpec((1, RB, D), lambda b, r: (b, r, 0)),
            pl.BlockSpec((1, RB, 1), lambda b, r: (b, r, 0)),
            pl.BlockSpec((1, RB, 1), lambda b, r: (b, r, 0)),
        ],
        out_shape=[
            jax.ShapeDtypeStruct((2, NP, D), jnp.float32),
            jax.ShapeDtypeStruct((2, NP, 1), jnp.float32),
            jax.ShapeDtypeStruct((2, NP, 1), jnp.float32),
        ],
    )(numer, denom, asv, adv, h, b1, W, av, ad)


def _fin_body(num_ref, den_ref, asv_ref, adv_ref, h_ref, b2_ref, wm1_ref,
              bm1_ref, wm2_ref, bm2_ref, batch_ref, o_ref, p_acc):
    b = pl.program_id(0)
    r = pl.program_id(1)
    el = jnp.exp(_leaky(asv_ref[0] + adv_ref[0]))
    den = jnp.sum(den_ref[0], axis=0)
    x = (num_ref[0] + el * h_ref[0]) / (den + el + 1e-16) + b2_ref[...]
    x = _elu(x)
    y = jnp.maximum(jnp.dot(x, wm1_ref[...], preferred_element_type=jnp.float32)
                    + bm1_ref[...], 0.0)
    y = jnp.dot(y, wm2_ref[...], preferred_element_type=jnp.float32) + bm2_ref[...]
    gid = lax.broadcasted_iota(jnp.int32, (RB, D), 1)
    oh = jnp.where((gid == batch_ref[0]) & (gid < NGRAPH), 1.0, 0.0)
    p = lax.dot_general(oh, y, (((0,), (0,)), ((), ())),
                        preferred_element_type=jnp.float32)

    @pl.when(r == 0)
    def _init():
        p_acc[b] = p

    @pl.when(r > 0)
    def _acc():
        p_acc[b] += p

    @pl.when((b == 1) & (r == NRB - 1))
    def _emit():
        res = lax.dot_general(p_acc[0], p_acc[1], (((0,), (0,)), ((), ())),
                              preferred_element_type=jnp.float32)
        o_ref[...] = res[:NCLS, :NCLS]


def _tc_fin(numer, denom, asv, adv, h, b2, Wm1, bm1, Wm2p, bm2p, batches):
    return pl.pallas_call(
        _fin_body,
        grid=(2, NRB),
        in_specs=[
            pl.BlockSpec((1, RB, D), lambda b, r: (b, r, 0)),
            pl.BlockSpec((1, NTILE, RB, 1), lambda b, r: (b, 0, r, 0)),
            pl.BlockSpec((1, RB, 1), lambda b, r: (b, r, 0)),
            pl.BlockSpec((1, RB, 1), lambda b, r: (b, r, 0)),
            pl.BlockSpec((1, RB, D), lambda b, r: (b, r, 0)),
            pl.BlockSpec((1, D), lambda b, r: (0, 0)),
            pl.BlockSpec((D, D), lambda b, r: (0, 0)),
            pl.BlockSpec((1, D), lambda b, r: (0, 0)),
            pl.BlockSpec((D, D), lambda b, r: (0, 0)),
            pl.BlockSpec((1, D), lambda b, r: (0, 0)),
            pl.BlockSpec((1, RB, 1), lambda b, r: (b, r, 0)),
        ],
        out_specs=pl.BlockSpec((NCLS, NCLS), lambda b, r: (0, 0)),
        out_shape=jax.ShapeDtypeStruct((NCLS, NCLS), jnp.float32),
        scratch_shapes=[pltpu.VMEM((2, D, D), jnp.float32)],
    )(numer, denom, asv, adv, h, b2, Wm1, bm1, Wm2p, bm2p, batches)


_MESH = plsc.VectorSubcoreMesh(core_axis_name="c", subcore_axis_name="s")


def _edge_body(h_hbm, asv_hbm, adv_hbm, srcI_hbm, dstI_hbm,
               numer_hbm, denom_hbm,
               numer_sh,
               src_v, dst_v, asv_v, adv_v, dloc_v, rows_a, rows_b, rows_c,
               rows_d, coef_v, sem_g, sem_s):
    b = lax.axis_index("c")
    t = lax.axis_index("s")
    rows = (rows_a, rows_b, rows_c, rows_d)

    z16 = jnp.zeros((16,), jnp.float32)

    def _zd(i, c):
        for r in range(8):
            dloc_v[i, pl.ds(r * 16, 16)] = z16
        return c
    lax.fori_loop(0, DNW, _zd, 0)

    def _zr(i, c):
        for r in range(8):
            rows_a[i, pl.ds(r * 16, 16)] = z16
        return c
    lax.fori_loop(0, CH, _zr, 0)

    for j in range(ROWS_PT // 8):
        pltpu.sync_copy(rows_a.at[pl.ds(0, 8)],
                        numer_sh.at[pl.ds(t * ROWS_PT + j * 8, 8)])

    pltpu.sync_copy(asv_hbm.at[b], asv_v)
    pltpu.sync_copy(adv_hbm.at[b], adv_v)

    plsc.subcore_barrier()

    def _sweep(g, carry):
        pltpu.sync_copy(srcI_hbm.at[b, t].at[pl.ds(g * GRP, GRP)], src_v)
        pltpu.sync_copy(dstI_hbm.at[b, t].at[pl.ds(g * GRP, GRP)], dst_v)
        gathers = [None] * GRP
        scatters = [None] * GRP
        for c in range(3):
            gathers[c] = pltpu.async_copy(
                h_hbm.at[b].at[src_v.at[c]], rows[c % 4], sem_g)
        for c in range(GRP):
            i = c % 4
            if c + 3 < GRP:
                if scatters[c - 1] is not None:
                    scatters[c - 1].wait()
                gathers[c + 3] = pltpu.async_copy(
                    h_hbm.at[b].at[src_v.at[c + 3]], rows[(c + 3) % 4], sem_g)
            for k in range(CH // 16):
                s16 = src_v[c, pl.ds(k * 16, 16)]
                d16 = dst_v[c, pl.ds(k * 16, 16)]
                sr = lax.shift_right_logical(s16, 7)
                sc = lax.bitwise_and(s16, 127)
                dr = lax.shift_right_logical(d16, 7)
                dc = lax.bitwise_and(d16, 127)
                a = (plsc.load_gather(asv_v, [sr, sc])
                     + plsc.load_gather(adv_v, [dr, dc]))
                e = jnp.exp(jnp.where(a > 0, a, a * 0.2))
                plsc.addupdate_scatter(dloc_v, [dr, dc], e)
                coef_v[pl.ds(k * 16, 16)] = e
            gathers[c].wait()

            rv = rows[i]

            def _scale(jj, cc, rv=rv):
                cjv = plsc.load_gather(coef_v, [jnp.full((16,), jj, jnp.int32)])
                for r in range(8):
                    rv[jj, pl.ds(r * 16, 16)] = rv[jj, pl.ds(r * 16, 16)] * cjv
                return cc
            lax.fori_loop(0, CH, _scale, 0)

            scatters[c] = pltpu.async_copy(
                rows[i], numer_sh.at[dst_v.at[c]], sem_s, add=True)
        for c in range(GRP - 4, GRP):
            scatters[c].wait()
        return carry
    lax.fori_loop(0, NGROUP, _sweep, 0)

    pltpu.sync_copy(dloc_v, denom_hbm.at[b, t])

    plsc.subcore_barrier()

    pltpu.sync_copy(numer_sh.at[pl.ds(t * ROWS_PT, ROWS_PT)],
                    numer_hbm.at[b].at[pl.ds(t * ROWS_PT, ROWS_PT)])


_sc_edge = pl.kernel(
    _edge_body,
    mesh=_MESH,
    compiler_params=pltpu.CompilerParams(needs_layout_passes=False),
    out_type=[
        jax.ShapeDtypeStruct((2, NSC, D), jnp.float32),
        jax.ShapeDtypeStruct((2, NTILE, DNW, 128), jnp.float32),
    ],
    scratch_types=[
        pltpu.VMEM_SHARED((NSC, D), jnp.float32),
        pltpu.VMEM((GRP, CH), jnp.int32),
        pltpu.VMEM((GRP, CH), jnp.int32),
        pltpu.VMEM((DNW, 128), jnp.float32),
        pltpu.VMEM((DNW, 128), jnp.float32),
        pltpu.VMEM((DNW, 128), jnp.float32),
        pltpu.VMEM((CH, D), jnp.float32),
        pltpu.VMEM((CH, D), jnp.float32),
        pltpu.VMEM((CH, D), jnp.float32),
        pltpu.VMEM((CH, D), jnp.float32),
        pltpu.VMEM((CH,), jnp.float32),
        pltpu.SemaphoreType.DMA,
        pltpu.SemaphoreType.DMA,
    ],
)


def _pad_nodes(x):
    return jnp.concatenate(
        [x, jnp.zeros((NP - NN, x.shape[1]), x.dtype)], axis=0)


def _prep_edges(edge_index):
    pad_s = jnp.full((NTILE * EPT - E,), NN, jnp.int32)
    pad_d = jnp.zeros((NTILE * EPT - E,), jnp.int32)
    src = jnp.concatenate([edge_index[0], pad_s]).reshape(NTILE, NCHUNK, CH)
    dst = jnp.concatenate([edge_index[1], pad_d]).reshape(NTILE, NCHUNK, CH)
    return src, dst


def _adv_tab(adv):
    t = adv.reshape(2, NP)
    t = jnp.concatenate([t, jnp.zeros((2, DNW * 128 - NP), jnp.float32)], 1)
    return t.reshape(2, DNW, 128)


def _sc_tables(asv):
    t = asv.reshape(2, NP)
    t = jnp.concatenate([t, jnp.zeros((2, DNW * 128 - NP), jnp.float32)], 1)
    t = jnp.where(jnp.arange(DNW * 128)[None, :] >= NN, -1e30, t)
    return t.reshape(2, DNW, 128)


def kernel(x1, edge_index1, batch1, x2, edge_index2, batch2,
           Wc1, as1, ad1, bc1, Wc2, as2, ad2, bc2, Wm1, bm1, Wm2, bm2):
    xs = jnp.stack([_pad_nodes(x1), _pad_nodes(x2)])
    s1, d1 = _prep_edges(edge_index1)
    s2, d2 = _prep_edges(edge_index2)
    srcI = jnp.stack([s1, s2])
    dstI = jnp.stack([d1, d2])
    bpad = jnp.full((NP - NN,), NGRAPH, jnp.int32)
    batches = jnp.stack([jnp.concatenate([batch1, bpad]),
                         jnp.concatenate([batch2, bpad])])[..., None]

    as1r, ad1r = as1.reshape(1, D), ad1.reshape(1, D)
    as2r, ad2r = as2.reshape(1, D), ad2.reshape(1, D)
    bc1r, bc2r = bc1.reshape(1, D), bc2.reshape(1, D)
    bm1r = bm1.reshape(1, D)
    Wm2p = jnp.zeros((D, D), jnp.float32).at[:, :NCLS].set(Wm2)
    bm2p = jnp.zeros((1, D), jnp.float32).at[0, :NCLS].set(bm2)

    zpad = ((0, 0), (0, NP - NSC), (0, 0))

    def _dn(dp):
        return dp.reshape(2, NTILE, NP, 1)

    h1, asv1, adv1 = _tc_dense1(xs, Wc1, as1r, ad1r)
    numer1, denom1 = _sc_edge(h1, _sc_tables(asv1), _adv_tab(adv1),
                              srcI, dstI)
    h2, asv2, adv2 = _tc_mid(jnp.pad(numer1, zpad), _dn(denom1), asv1, adv1,
                             h1, bc1r, Wc2, as2r, ad2r)
    numer2, denom2 = _sc_edge(h2, _sc_tables(asv2), _adv_tab(adv2),
                              srcI, dstI)
    return _tc_fin(jnp.pad(numer2, zpad), _dn(denom2), asv2, adv2, h2, bc2r,
                   Wm1, bm1r, Wm2p, bm2p, batches)


# speedup vs baseline: 1.0947x; 1.0947x over previous
import functools

import jax
import jax.numpy as jnp
from jax import lax
from jax.experimental import pallas as pl
from jax.experimental.pallas import tpu as pltpu
from jax.experimental.pallas import tpu_sc as plsc

NN = 10000
NP = 10240
D = 128
NCLS = 16
NGRAPH = 64
E = 320000
NTILE = 16
CH = 64
GRP = 8
NCHUNK = 320
NGROUP = NCHUNK // GRP
EPT = NCHUNK * CH
DNW = NP // 128
NSC = 10112
ROWS_PT = NSC // NTILE


def _leaky(x):
    return jnp.where(x > 0, x, 0.2 * x)


def _elu(x):
    return jnp.where(x > 0, x, jnp.exp(jnp.minimum(x, 0.0)) - 1.0)


RB = 2048
NRB = NP // RB


def _dense1_body(x_ref, w_ref, av_ref, ad_ref, h_ref, asv_ref, adv_ref):
    h = jnp.dot(x_ref[0], w_ref[...], preferred_element_type=jnp.float32)
    h_ref[0] = h
    asv_ref[0] = jnp.sum(h * av_ref[...], axis=-1, keepdims=True)
    adv_ref[0] = jnp.sum(h * ad_ref[...], axis=-1, keepdims=True)


def _tc_dense1(xs, W, av, ad):
    return pl.pallas_call(
        _dense1_body,
        grid=(2, NRB),
        in_specs=[
            pl.BlockSpec((1, RB, D), lambda b, r: (b, r, 0)),
            pl.BlockSpec((D, D), lambda b, r: (0, 0)),
            pl.BlockSpec((1, D), lambda b, r: (0, 0)),
            pl.BlockSpec((1, D), lambda b, r: (0, 0)),
        ],
        out_specs=[
            pl.BlockSpec((1, RB, D), lambda b, r: (b, r, 0)),
            pl.BlockSpec((1, RB, 1), lambda b, r: (b, r, 0)),
            pl.BlockSpec((1, RB, 1), lambda b, r: (b, r, 0)),
        ],
        out_shape=[
            jax.ShapeDtypeStruct((2, NP, D), jnp.float32),
            jax.ShapeDtypeStruct((2, NP, 1), jnp.float32),
            jax.ShapeDtypeStruct((2, NP, 1), jnp.float32),
        ],
    )(xs, W, av, ad)


def _mid_body(num_ref, den_ref, asv_ref, adv_ref, h_ref, b1_ref, w_ref,
              av_ref, ad_ref, h2_ref, asv2_ref, adv2_ref):
    el = jnp.exp(_leaky(asv_ref[0] + adv_ref[0]))
    den = jnp.sum(den_ref[0], axis=0)
    x = (num_ref[0] + el * h_ref[0]) / (den + el + 1e-16) + b1_ref[...]
    x = _elu(x)
    h2 = jnp.dot(x, w_ref[...], preferred_element_type=jnp.float32)
    h2_ref[0] = h2
    asv2_ref[0] = jnp.sum(h2 * av_ref[...], axis=-1, keepdims=True)
    adv2_ref[0] = jnp.sum(h2 * ad_ref[...], axis=-1, keepdims=True)


def _tc_mid(numer, denom, asv, adv, h, b1, W, av, ad):
    return pl.pallas_call(
        _mid_body,
        grid=(2, NRB),
        in_specs=[
            pl.BlockSpec((1, RB, D), lambda b, r: (b, r, 0)),
            pl.BlockSpec((1, NTILE, RB, 1), lambda b, r: (b, 0, r, 0)),
            pl.BlockSpec((1, RB, 1), lambda b, r: (b, r, 0)),
            pl.BlockSpec((1, RB, 1), lambda b, r: (b, r, 0)),
            pl.BlockSpec((1, RB, D), lambda b, r: (b, r, 0)),
            pl.BlockSpec((1, D), lambda b, r: (0, 0)),
            pl.BlockSpec((D, D), lambda b, r: (0, 0)),
            pl.BlockSpec((1, D), lambda b, r: (0, 0)),
            pl.BlockSpec((1, D), lambda b, r: (0, 0)),
        ],
        out_specs=[
            pl.BlockSpec((1, RB, D), lambda b, r: (b, r, 0)),
            pl.BlockSpec((1, RB, 1), lambda b, r: (b, r, 0)),
            pl.BlockSpec((1, RB, 1), lambda b, r: (b, r, 0)),
        ],
        out_shape=[
            jax.ShapeDtypeStruct((2, NP, D), jnp.float32),
            jax.ShapeDtypeStruct((2, NP, 1), jnp.float32),
            jax.ShapeDtypeStruct((2, NP, 1), jnp.float32),
        ],
    )(numer, denom, asv, adv, h, b1, W, av, ad)


def _fin_body(num_ref, den_ref, asv_ref, adv_ref, h_ref, b2_ref, wm1_ref,
              bm1_ref, wm2_ref, bm2_ref, batch_ref, o_ref, p_acc):
    b = pl.program_id(0)
    r = pl.program_id(1)
    el = jnp.exp(_leaky(asv_ref[0] + adv_ref[0]))
    den = jnp.sum(den_ref[0], axis=0)
    x = (num_ref[0] + el * h_ref[0]) / (den + el + 1e-16) + b2_ref[...]
    x = _elu(x)
    y = jnp.maximum(jnp.dot(x, wm1_ref[...], preferred_element_type=jnp.float32)
                    + bm1_ref[...], 0.0)
    y = jnp.dot(y, wm2_ref[...], preferred_element_type=jnp.float32) + bm2_ref[...]
    gid = lax.broadcasted_iota(jnp.int32, (RB, D), 1)
    oh = jnp.where((gid == batch_ref[0]) & (gid < NGRAPH), 1.0, 0.0)
    p = lax.dot_general(oh, y, (((0,), (0,)), ((), ())),
                        preferred_element_type=jnp.float32)

    @pl.when(r == 0)
    def _init():
        p_acc[b] = p

    @pl.when(r > 0)
    def _acc():
        p_acc[b] += p

    @pl.when((b == 1) & (r == NRB - 1))
    def _emit():
        res = lax.dot_general(p_acc[0], p_acc[1], (((0,), (0,)), ((), ())),
                              preferred_element_type=jnp.float32)
        o_ref[...] = res[:NCLS, :NCLS]


def _tc_fin(numer, denom, asv, adv, h, b2, Wm1, bm1, Wm2p, bm2p, batches):
    return pl.pallas_call(
        _fin_body,
        grid=(2, NRB),
        in_specs=[
            pl.BlockSpec((1, RB, D), lambda b, r: (b, r, 0)),
            pl.BlockSpec((1, NTILE, RB, 1), lambda b, r: (b, 0, r, 0)),
            pl.BlockSpec((1, RB, 1), lambda b, r: (b, r, 0)),
            pl.BlockSpec((1, RB, 1), lambda b, r: (b, r, 0)),
            pl.BlockSpec((1, RB, D), lambda b, r: (b, r, 0)),
            pl.BlockSpec((1, D), lambda b, r: (0, 0)),
            pl.BlockSpec((D, D), lambda b, r: (0, 0)),
            pl.BlockSpec((1, D), lambda b, r: (0, 0)),
            pl.BlockSpec((D, D), lambda b, r: (0, 0)),
            pl.BlockSpec((1, D), lambda b, r: (0, 0)),
            pl.BlockSpec((1, RB, 1), lambda b, r: (b, r, 0)),
        ],
        out_specs=pl.BlockSpec((NCLS, NCLS), lambda b, r: (0, 0)),
        out_shape=jax.ShapeDtypeStruct((NCLS, NCLS), jnp.float32),
        scratch_shapes=[pltpu.VMEM((2, D, D), jnp.float32)],
    )(numer, denom, asv, adv, h, b2, Wm1, bm1, Wm2p, bm2p, batches)


_MESH = plsc.VectorSubcoreMesh(core_axis_name="c", subcore_axis_name="s")


def _edge_body(h_hbm, asv_hbm, adv_hbm, srcI_hbm, dstI_hbm,
               numer_hbm, denom_hbm,
               numer_sh,
               src_v, dst_v, asv_v, adv_v, dloc_v, rows_a, rows_b,
               coef_v, sem_g, sem_g2, sem_s):
    b = lax.axis_index("c")
    t = lax.axis_index("s")
    rows = (rows_a, rows_b)

    z16 = jnp.zeros((16,), jnp.float32)

    def _zd(i, c):
        for r in range(8):
            dloc_v[i, pl.ds(r * 16, 16)] = z16
        return c
    lax.fori_loop(0, DNW, _zd, 0)

    def _zr(i, c):
        for r in range(8):
            rows_a[i, pl.ds(r * 16, 16)] = z16
        return c
    lax.fori_loop(0, CH, _zr, 0)

    for j in range(ROWS_PT // 8):
        pltpu.sync_copy(rows_a.at[pl.ds(0, 8)],
                        numer_sh.at[pl.ds(t * ROWS_PT + j * 8, 8)])

    pltpu.sync_copy(asv_hbm.at[b], asv_v)
    pltpu.sync_copy(adv_hbm.at[b], adv_v)

    plsc.subcore_barrier()

    def _sweep(g, carry):
        pltpu.sync_copy(srcI_hbm.at[b, t].at[pl.ds(g * GRP, GRP)], src_v)
        pltpu.sync_copy(dstI_hbm.at[b, t].at[pl.ds(g * GRP, GRP)], dst_v)
        gathers = [None, None]
        scatters = [None, None]
        def _gissue(c, buf):
            g1 = pltpu.async_copy(
                h_hbm.at[b].at[src_v.at[c].at[pl.ds(0, CH // 2)]],
                buf.at[pl.ds(0, CH // 2)], sem_g)
            g2 = pltpu.async_copy(
                h_hbm.at[b].at[src_v.at[c].at[pl.ds(CH // 2, CH // 2)]],
                buf.at[pl.ds(CH // 2, CH // 2)], sem_g2)
            return (g1, g2)

        gathers[0] = _gissue(0, rows[0])
        for c in range(GRP):
            i = c % 2
            if c + 1 < GRP:
                j = (c + 1) % 2
                if scatters[j] is not None:
                    scatters[j].wait()
                gathers[j] = _gissue(c + 1, rows[j])
            for k in range(CH // 16):
                s16 = src_v[c, pl.ds(k * 16, 16)]
                d16 = dst_v[c, pl.ds(k * 16, 16)]
                sr = lax.shift_right_logical(s16, 7)
                sc = lax.bitwise_and(s16, 127)
                dr = lax.shift_right_logical(d16, 7)
                dc = lax.bitwise_and(d16, 127)
                a = (plsc.load_gather(asv_v, [sr, sc])
                     + plsc.load_gather(adv_v, [dr, dc]))
                e = jnp.exp(jnp.where(a > 0, a, a * 0.2))
                plsc.addupdate_scatter(dloc_v, [dr, dc], e)
                coef_v[pl.ds(k * 16, 16)] = e
            gathers[i][0].wait()
            gathers[i][1].wait()

            rv = rows[i]

            def _scale(jj, cc, rv=rv):
                cjv = plsc.load_gather(coef_v, [jnp.full((16,), jj, jnp.int32)])
                for r in range(8):
                    rv[jj, pl.ds(r * 16, 16)] = rv[jj, pl.ds(r * 16, 16)] * cjv
                return cc
            lax.fori_loop(0, CH, _scale, 0)

            scatters[i] = pltpu.async_copy(
                rows[i], numer_sh.at[dst_v.at[c]], sem_s, add=True)
        scatters[0].wait()
        scatters[1].wait()
        return carry
    lax.fori_loop(0, NGROUP, _sweep, 0)

    pltpu.sync_copy(dloc_v, denom_hbm.at[b, t])

    plsc.subcore_barrier()

    pltpu.sync_copy(numer_sh.at[pl.ds(t * ROWS_PT, ROWS_PT)],
                    numer_hbm.at[b].at[pl.ds(t * ROWS_PT, ROWS_PT)])


_sc_edge = pl.kernel(
    _edge_body,
    mesh=_MESH,
    compiler_params=pltpu.CompilerParams(needs_layout_passes=False),
    out_type=[
        jax.ShapeDtypeStruct((2, NSC, D), jnp.float32),
        jax.ShapeDtypeStruct((2, NTILE, DNW, 128), jnp.float32),
    ],
    scratch_types=[
        pltpu.VMEM_SHARED((NSC, D), jnp.float32),
        pltpu.VMEM((GRP, CH), jnp.int32),
        pltpu.VMEM((GRP, CH), jnp.int32),
        pltpu.VMEM((DNW, 128), jnp.float32),
        pltpu.VMEM((DNW, 128), jnp.float32),
        pltpu.VMEM((DNW, 128), jnp.float32),
        pltpu.VMEM((CH, D), jnp.float32),
        pltpu.VMEM((CH, D), jnp.float32),
        pltpu.VMEM((CH,), jnp.float32),
        pltpu.SemaphoreType.DMA,
        pltpu.SemaphoreType.DMA,
        pltpu.SemaphoreType.DMA,
    ],
)


def _pad_nodes(x):
    return jnp.concatenate(
        [x, jnp.zeros((NP - NN, x.shape[1]), x.dtype)], axis=0)


def _prep_edges(edge_index):
    pad_s = jnp.full((NTILE * EPT - E,), NN, jnp.int32)
    pad_d = jnp.zeros((NTILE * EPT - E,), jnp.int32)
    src = jnp.concatenate([edge_index[0], pad_s]).reshape(NTILE, NCHUNK, CH)
    dst = jnp.concatenate([edge_index[1], pad_d]).reshape(NTILE, NCHUNK, CH)
    return src, dst


def _adv_tab(adv):
    t = adv.reshape(2, NP)
    t = jnp.concatenate([t, jnp.zeros((2, DNW * 128 - NP), jnp.float32)], 1)
    return t.reshape(2, DNW, 128)


def _sc_tables(asv):
    t = asv.reshape(2, NP)
    t = jnp.concatenate([t, jnp.zeros((2, DNW * 128 - NP), jnp.float32)], 1)
    t = jnp.where(jnp.arange(DNW * 128)[None, :] >= NN, -1e30, t)
    return t.reshape(2, DNW, 128)


def kernel(x1, edge_index1, batch1, x2, edge_index2, batch2,
           Wc1, as1, ad1, bc1, Wc2, as2, ad2, bc2, Wm1, bm1, Wm2, bm2):
    xs = jnp.stack([_pad_nodes(x1), _pad_nodes(x2)])
    s1, d1 = _prep_edges(edge_index1)
    s2, d2 = _prep_edges(edge_index2)
    srcI = jnp.stack([s1, s2])
    dstI = jnp.stack([d1, d2])
    bpad = jnp.full((NP - NN,), NGRAPH, jnp.int32)
    batches = jnp.stack([jnp.concatenate([batch1, bpad]),
                         jnp.concatenate([batch2, bpad])])[..., None]

    as1r, ad1r = as1.reshape(1, D), ad1.reshape(1, D)
    as2r, ad2r = as2.reshape(1, D), ad2.reshape(1, D)
    bc1r, bc2r = bc1.reshape(1, D), bc2.reshape(1, D)
    bm1r = bm1.reshape(1, D)
    Wm2p = jnp.zeros((D, D), jnp.float32).at[:, :NCLS].set(Wm2)
    bm2p = jnp.zeros((1, D), jnp.float32).at[0, :NCLS].set(bm2)

    zpad = ((0, 0), (0, NP - NSC), (0, 0))

    def _dn(dp):
        return dp.reshape(2, NTILE, NP, 1)

    h1, asv1, adv1 = _tc_dense1(xs, Wc1, as1r, ad1r)
    numer1, denom1 = _sc_edge(h1, _sc_tables(asv1), _adv_tab(adv1),
                              srcI, dstI)
    h2, asv2, adv2 = _tc_mid(jnp.pad(numer1, zpad), _dn(denom1), asv1, adv1,
                             h1, bc1r, Wc2, as2r, ad2r)
    numer2, denom2 = _sc_edge(h2, _sc_tables(asv2), _adv_tab(adv2),
                              srcI, dstI)
    return _tc_fin(jnp.pad(numer2, zpad), _dn(denom2), asv2, adv2, h2, bc2r,
                   Wm1, bm1r, Wm2p, bm2p, batches)


# speedup vs baseline: 1.0965x; 1.0017x over previous
import functools

import jax
import jax.numpy as jnp
from jax import lax
from jax.experimental import pallas as pl
from jax.experimental.pallas import tpu as pltpu
from jax.experimental.pallas import tpu_sc as plsc

NN = 10000
NP = 10240
D = 128
NCLS = 16
NGRAPH = 64
E = 320000
NTILE = 16
CH = 64
GRP = 8
NCHUNK = 320
NGROUP = NCHUNK // GRP
EPT = NCHUNK * CH
DNW = NP // 128
NSC = 10112
ROWS_PT = NSC // NTILE


def _leaky(x):
    return jnp.where(x > 0, x, 0.2 * x)


def _elu(x):
    return jnp.where(x > 0, x, jnp.exp(jnp.minimum(x, 0.0)) - 1.0)


RB = 2048
NRB = NP // RB


def _dense1_body(x_ref, w_ref, av_ref, ad_ref, h_ref, asv_ref, adv_ref):
    h = jnp.dot(x_ref[0], w_ref[...], preferred_element_type=jnp.float32)
    h_ref[0] = h
    asv_ref[0] = jnp.sum(h * av_ref[...], axis=-1, keepdims=True)
    adv_ref[0] = jnp.sum(h * ad_ref[...], axis=-1, keepdims=True)


def _tc_dense1(xs, W, av, ad):
    return pl.pallas_call(
        _dense1_body,
        grid=(2, NRB),
        in_specs=[
            pl.BlockSpec((1, RB, D), lambda b, r: (b, r, 0)),
            pl.BlockSpec((D, D), lambda b, r: (0, 0)),
            pl.BlockSpec((1, D), lambda b, r: (0, 0)),
            pl.BlockSpec((1, D), lambda b, r: (0, 0)),
        ],
        out_specs=[
            pl.BlockSpec((1, RB, D), lambda b, r: (b, r, 0)),
            pl.BlockSpec((1, RB, 1), lambda b, r: (b, r, 0)),
            pl.BlockSpec((1, RB, 1), lambda b, r: (b, r, 0)),
        ],
        out_shape=[
            jax.ShapeDtypeStruct((2, NP, D), jnp.float32),
            jax.ShapeDtypeStruct((2, NP, 1), jnp.float32),
            jax.ShapeDtypeStruct((2, NP, 1), jnp.float32),
        ],
    )(xs, W, av, ad)


def _mid_body(num_ref, den_ref, asv_ref, adv_ref, h_ref, b1_ref, w_ref,
              av_ref, ad_ref, h2_ref, asv2_ref, adv2_ref):
    el = jnp.exp(_leaky(asv_ref[0] + adv_ref[0]))
    den = jnp.sum(den_ref[0], axis=0)
    x = (num_ref[0] + el * h_ref[0]) / (den + el + 1e-16) + b1_ref[...]
    x = _elu(x)
    h2 = jnp.dot(x, w_ref[...], preferred_element_type=jnp.float32)
    h2_ref[0] = h2
    asv2_ref[0] = jnp.sum(h2 * av_ref[...], axis=-1, keepdims=True)
    adv2_ref[0] = jnp.sum(h2 * ad_ref[...], axis=-1, keepdims=True)


def _tc_mid(numer, denom, asv, adv, h, b1, W, av, ad):
    return pl.pallas_call(
        _mid_body,
        grid=(2, NRB),
        in_specs=[
            pl.BlockSpec((1, RB, D), lambda b, r: (b, r, 0)),
            pl.BlockSpec((1, NTILE, RB, 1), lambda b, r: (b, 0, r, 0)),
            pl.BlockSpec((1, RB, 1), lambda b, r: (b, r, 0)),
            pl.BlockSpec((1, RB, 1), lambda b, r: (b, r, 0)),
            pl.BlockSpec((1, RB, D), lambda b, r: (b, r, 0)),
            pl.BlockSpec((1, D), lambda b, r: (0, 0)),
            pl.BlockSpec((D, D), lambda b, r: (0, 0)),
            pl.BlockSpec((1, D), lambda b, r: (0, 0)),
            pl.BlockSpec((1, D), lambda b, r: (0, 0)),
        ],
        out_specs=[
            pl.BlockSpec((1, RB, D), lambda b, r: (b, r, 0)),
            pl.BlockSpec((1, RB, 1), lambda b, r: (b, r, 0)),
            pl.BlockSpec((1, RB, 1), lambda b, r: (b, r, 0)),
        ],
        out_shape=[
            jax.ShapeDtypeStruct((2, NP, D), jnp.float32),
            jax.ShapeDtypeStruct((2, NP, 1), jnp.float32),
            jax.ShapeDtypeStruct((2, NP, 1), jnp.float32),
        ],
    )(numer, denom, asv, adv, h, b1, W, av, ad)


def _fin_body(num_ref, den_ref, asv_ref, adv_ref, h_ref, b2_ref, wm1_ref,
              bm1_ref, wm2_ref, bm2_ref, batch_ref, o_ref, p_acc):
    b = pl.program_id(0)
    r = pl.program_id(1)
    el = jnp.exp(_leaky(asv_ref[0] + adv_ref[0]))
    den = jnp.sum(den_ref[0], axis=0)
    x = (num_ref[0] + el * h_ref[0]) / (den + el + 1e-16) + b2_ref[...]
    x = _elu(x)
    y = jnp.maximum(jnp.dot(x, wm1_ref[...], preferred_element_type=jnp.float32)
                    + bm1_ref[...], 0.0)
    y = jnp.dot(y, wm2_ref[...], preferred_element_type=jnp.float32) + bm2_ref[...]
    gid = lax.broadcasted_iota(jnp.int32, (RB, D), 1)
    oh = jnp.where((gid == batch_ref[0]) & (gid < NGRAPH), 1.0, 0.0)
    p = lax.dot_general(oh, y, (((0,), (0,)), ((), ())),
                        preferred_element_type=jnp.float32)

    @pl.when(r == 0)
    def _init():
        p_acc[b] = p

    @pl.when(r > 0)
    def _acc():
        p_acc[b] += p

    @pl.when((b == 1) & (r == NRB - 1))
    def _emit():
        res = lax.dot_general(p_acc[0], p_acc[1], (((0,), (0,)), ((), ())),
                              preferred_element_type=jnp.float32)
        o_ref[...] = res[:NCLS, :NCLS]


def _tc_fin(numer, denom, asv, adv, h, b2, Wm1, bm1, Wm2p, bm2p, batches):
    return pl.pallas_call(
        _fin_body,
        grid=(2, NRB),
        in_specs=[
            pl.BlockSpec((1, RB, D), lambda b, r: (b, r, 0)),
            pl.BlockSpec((1, NTILE, RB, 1), lambda b, r: (b, 0, r, 0)),
            pl.BlockSpec((1, RB, 1), lambda b, r: (b, r, 0)),
            pl.BlockSpec((1, RB, 1), lambda b, r: (b, r, 0)),
            pl.BlockSpec((1, RB, D), lambda b, r: (b, r, 0)),
            pl.BlockSpec((1, D), lambda b, r: (0, 0)),
            pl.BlockSpec((D, D), lambda b, r: (0, 0)),
            pl.BlockSpec((1, D), lambda b, r: (0, 0)),
            pl.BlockSpec((D, D), lambda b, r: (0, 0)),
            pl.BlockSpec((1, D), lambda b, r: (0, 0)),
            pl.BlockSpec((1, RB, 1), lambda b, r: (b, r, 0)),
        ],
        out_specs=pl.BlockSpec((NCLS, NCLS), lambda b, r: (0, 0)),
        out_shape=jax.ShapeDtypeStruct((NCLS, NCLS), jnp.float32),
        scratch_shapes=[pltpu.VMEM((2, D, D), jnp.float32)],
    )(numer, denom, asv, adv, h, b2, Wm1, bm1, Wm2p, bm2p, batches)


_MESH = plsc.VectorSubcoreMesh(core_axis_name="c", subcore_axis_name="s")


def _edge_body(h_hbm, asv_hbm, adv_hbm, srcI_hbm, dstI_hbm,
               numer_hbm, denom_hbm,
               numer_sh,
               src_v, dst_v, asv_v, adv_v, dloc_v, rows_a, rows_b,
               coef_v, sem_g, sem_s):
    b = lax.axis_index("c")
    t = lax.axis_index("s")
    rows = (rows_a, rows_b)

    z16 = jnp.zeros((16,), jnp.float32)

    def _zd(i, c):
        for r in range(8):
            dloc_v[i, pl.ds(r * 16, 16)] = z16
        return c
    lax.fori_loop(0, DNW, _zd, 0)

    def _zr(i, c):
        for r in range(8):
            rows_a[i, pl.ds(r * 16, 16)] = z16
        return c
    lax.fori_loop(0, CH, _zr, 0)

    for j in range(ROWS_PT // 8):
        pltpu.sync_copy(rows_a.at[pl.ds(0, 8)],
                        numer_sh.at[pl.ds(t * ROWS_PT + j * 8, 8)])

    pltpu.sync_copy(asv_hbm.at[b], asv_v)
    pltpu.sync_copy(adv_hbm.at[b], adv_v)

    plsc.subcore_barrier()

    def _sweep(g, carry):
        pltpu.sync_copy(srcI_hbm.at[b, t].at[pl.ds(g * GRP, GRP)], src_v)
        pltpu.sync_copy(dstI_hbm.at[b, t].at[pl.ds(g * GRP, GRP)], dst_v)
        gathers = [None, None]
        scatters = [None, None]
        gathers[0] = pltpu.async_copy(h_hbm.at[b].at[src_v.at[0]],
                                      rows[0], sem_g)
        for c in range(GRP):
            i = c % 2
            if c + 1 < GRP:
                j = (c + 1) % 2
                if scatters[j] is not None:
                    scatters[j].wait()
                gathers[j] = pltpu.async_copy(
                    h_hbm.at[b].at[src_v.at[c + 1]], rows[j], sem_g)
            for k in range(CH // 16):
                s16 = src_v[c, pl.ds(k * 16, 16)]
                d16 = dst_v[c, pl.ds(k * 16, 16)]
                sr = lax.shift_right_logical(s16, 7)
                sc = lax.bitwise_and(s16, 127)
                dr = lax.shift_right_logical(d16, 7)
                dc = lax.bitwise_and(d16, 127)
                a = (plsc.load_gather(asv_v, [sr, sc])
                     + plsc.load_gather(adv_v, [dr, dc]))
                e = jnp.exp(jnp.where(a > 0, a, a * 0.2))
                plsc.addupdate_scatter(dloc_v, [dr, dc], e)
                coef_v[pl.ds(k * 16, 16)] = e
            gathers[i].wait()

            rv = rows[i]

            def _scale(jj, cc, rv=rv):
                cjv = plsc.load_gather(coef_v, [jnp.full((16,), jj, jnp.int32)])
                for r in range(8):
                    rv[jj, pl.ds(r * 16, 16)] = rv[jj, pl.ds(r * 16, 16)] * cjv
                return cc
            lax.fori_loop(0, CH, _scale, 0)

            scatters[i] = pltpu.async_copy(
                rows[i], numer_sh.at[dst_v.at[c]], sem_s, add=True)
        scatters[0].wait()
        scatters[1].wait()
        return carry
    lax.fori_loop(0, NGROUP, _sweep, 0)

    pltpu.sync_copy(dloc_v, denom_hbm.at[b, t])

    plsc.subcore_barrier()

    pltpu.sync_copy(numer_sh.at[pl.ds(t * ROWS_PT, ROWS_PT)],
                    numer_hbm.at[b].at[pl.ds(t * ROWS_PT, ROWS_PT)])


_sc_edge = pl.kernel(
    _edge_body,
    mesh=_MESH,
    compiler_params=pltpu.CompilerParams(needs_layout_passes=False),
    out_type=[
        jax.ShapeDtypeStruct((2, NSC, D), jnp.float32),
        jax.ShapeDtypeStruct((2, NTILE, DNW, 128), jnp.float32),
    ],
    scratch_types=[
        pltpu.VMEM_SHARED((NSC, D), jnp.float32),
        pltpu.VMEM((GRP, CH), jnp.int32),
        pltpu.VMEM((GRP, CH), jnp.int32),
        pltpu.VMEM((DNW, 128), jnp.float32),
        pltpu.VMEM((DNW, 128), jnp.float32),
        pltpu.VMEM((DNW, 128), jnp.float32),
        pltpu.VMEM((CH, D), jnp.float32),
        pltpu.VMEM((CH, D), jnp.float32),
        pltpu.VMEM((CH,), jnp.float32),
        pltpu.SemaphoreType.DMA,
        pltpu.SemaphoreType.DMA,
    ],
)


def _pad_nodes(x):
    return jnp.concatenate(
        [x, jnp.zeros((NP - NN, x.shape[1]), x.dtype)], axis=0)


def _prep_edges(edge_index):
    pad_s = jnp.full((NTILE * EPT - E,), NN, jnp.int32)
    pad_d = jnp.zeros((NTILE * EPT - E,), jnp.int32)
    src = jnp.concatenate([edge_index[0], pad_s]).reshape(NTILE, NCHUNK, CH)
    dst = jnp.concatenate([edge_index[1], pad_d]).reshape(NTILE, NCHUNK, CH)
    return src, dst


def _adv_tab(adv):
    t = adv.reshape(2, NP)
    t = jnp.concatenate([t, jnp.zeros((2, DNW * 128 - NP), jnp.float32)], 1)
    return t.reshape(2, DNW, 128)


def _sc_tables(asv):
    t = asv.reshape(2, NP)
    t = jnp.concatenate([t, jnp.zeros((2, DNW * 128 - NP), jnp.float32)], 1)
    t = jnp.where(jnp.arange(DNW * 128)[None, :] >= NN, -1e30, t)
    return t.reshape(2, DNW, 128)


def kernel(x1, edge_index1, batch1, x2, edge_index2, batch2,
           Wc1, as1, ad1, bc1, Wc2, as2, ad2, bc2, Wm1, bm1, Wm2, bm2):
    xs = jnp.stack([_pad_nodes(x1), _pad_nodes(x2)])
    s1, d1 = _prep_edges(edge_index1)
    s2, d2 = _prep_edges(edge_index2)
    srcI = jnp.stack([s1, s2])
    dstI = jnp.stack([d1, d2])
    bpad = jnp.full((NP - NN,), NGRAPH, jnp.int32)
    batches = jnp.stack([jnp.concatenate([batch1, bpad]),
                         jnp.concatenate([batch2, bpad])])[..., None]

    as1r, ad1r = as1.reshape(1, D), ad1.reshape(1, D)
    as2r, ad2r = as2.reshape(1, D), ad2.reshape(1, D)
    bc1r, bc2r = bc1.reshape(1, D), bc2.reshape(1, D)
    bm1r = bm1.reshape(1, D)
    Wm2p = jnp.zeros((D, D), jnp.float32).at[:, :NCLS].set(Wm2)
    bm2p = jnp.zeros((1, D), jnp.float32).at[0, :NCLS].set(bm2)

    zpad = ((0, 0), (0, NP - NSC), (0, 0))

    def _dn(dp):
        return dp.reshape(2, NTILE, NP, 1)

    h1, asv1, adv1 = _tc_dense1(xs, Wc1, as1r, ad1r)
    numer1, denom1 = _sc_edge(h1, _sc_tables(asv1), _adv_tab(adv1),
                              srcI, dstI)
    h2, asv2, adv2 = _tc_mid(jnp.pad(numer1, zpad), _dn(denom1), asv1, adv1,
                             h1, bc1r, Wc2, as2r, ad2r)
    numer2, denom2 = _sc_edge(h2, _sc_tables(asv2), _adv_tab(adv2),
                              srcI, dstI)
    return _tc_fin(jnp.pad(numer2, zpad), _dn(denom2), asv2, adv2, h2, bc2r,
                   Wm1, bm1r, Wm2p, bm2p, batches)
